# baseline (speedup 1.0000x reference)
import functools

import jax
import jax.numpy as jnp
from jax import lax
from jax.experimental import pallas as pl
from jax.experimental.pallas import tpu as pltpu
from jax.experimental.pallas import tpu_sc as plsc

NC = 2
NS = 16
NW = NC * NS
LANES = 16
IDX_PER_STREAM = 128


def _tree_sum(vals):
    while len(vals) > 1:
        nxt = []
        for i in range(0, len(vals) - 1, 2):
            nxt.append(vals[i] + vals[i + 1])
        if len(vals) % 2:
            nxt.append(vals[-1])
        vals = nxt
    return vals[0]


def _make_sc_kernel(B, L, V, D, CH):
    b_per_w = B // NW
    n_chunks = b_per_w // CH
    rows_per_chunk = CH * L
    steps_per_chunk = rows_per_chunk // IDX_PER_STREAM
    n_steps = n_chunks * steps_per_chunk
    cgroups = D // LANES
    inv_l = 1.0 / L

    mesh = plsc.VectorSubcoreMesh(core_axis_name="c", subcore_axis_name="s")

    @functools.partial(
        pl.kernel,
        mesh=mesh,
        out_type=jax.ShapeDtypeStruct((B, D), jnp.float32),
        compiler_params=pltpu.CompilerParams(use_tc_tiling_on_sc=False),
        scratch_types=[
            pltpu.VMEM((n_steps, IDX_PER_STREAM), jnp.int32),
            pltpu.VMEM((rows_per_chunk, D), jnp.float32),
            pltpu.VMEM((CH, D), jnp.float32),
            pltpu.SemaphoreType.DMA,
        ],
    )
    def sc_kernel(ctx_hbm, table_hbm, out_hbm, idx_v, rows_v, ob_v, sem):
        wid = lax.axis_index("s") * NC + lax.axis_index("c")
        pltpu.sync_copy(ctx_hbm.at[wid], idx_v)

        def chunk_body(c, carry):
            copies = []
            for s in range(steps_per_chunk):
                step = c * steps_per_chunk + s
                copies.append(
                    pltpu.async_copy(
                        table_hbm.at[idx_v.at[step]],
                        rows_v.at[pl.ds(s * IDX_PER_STREAM, IDX_PER_STREAM)],
                        sem,
                    )
                )
            for cp in copies:
                cp.wait()

            def e_body(e, carry2):
                base = e * L
                for cg in range(cgroups):
                    sl = pl.ds(cg * LANES, LANES)
                    vals = [rows_v[base + j, sl] for j in range(L)]
                    ob_v[e, sl] = _tree_sum(vals) * inv_l
                return carry2

            lax.fori_loop(0, CH, e_body, 0, unroll=False)
            out_base = wid * b_per_w + c * CH
            pltpu.sync_copy(ob_v, out_hbm.at[pl.ds(out_base, CH)])
            return carry

        lax.fori_loop(0, n_chunks, chunk_body, 0, unroll=False)

    return sc_kernel


@jax.jit
def kernel(contexts, table):
    B, L = contexts.shape
    V, D = table.shape
    CH = 32
    assert (CH * L) % IDX_PER_STREAM == 0
    assert B % (NW * CH) == 0
    ctx3 = contexts.reshape(NW, (B // NW * L) // IDX_PER_STREAM, IDX_PER_STREAM)
    return _make_sc_kernel(B, L, V, D, CH)(ctx3, table)


# speedup vs baseline: 1.0525x; 1.0525x over previous
import functools

import jax
import jax.numpy as jnp
from jax import lax
from jax.experimental import pallas as pl
from jax.experimental.pallas import tpu as pltpu
from jax.experimental.pallas import tpu_sc as plsc

NC = 2
NS = 16
NW = NC * NS
LANES = 16
IDX_PER_STREAM = 128


def _tree_sum(vals):
    while len(vals) > 1:
        nxt = []
        for i in range(0, len(vals) - 1, 2):
            nxt.append(vals[i] + vals[i + 1])
        if len(vals) % 2:
            nxt.append(vals[-1])
        vals = nxt
    return vals[0]


def _make_sc_kernel(B, L, V, D, CH):
    b_per_w = B // NW
    n_chunks = b_per_w // CH
    rows_per_chunk = CH * L
    steps_per_chunk = rows_per_chunk // IDX_PER_STREAM
    n_steps = n_chunks * steps_per_chunk
    cgroups = D // LANES
    inv_l = 1.0 / L

    mesh = plsc.VectorSubcoreMesh(core_axis_name="c", subcore_axis_name="s")

    @functools.partial(
        pl.kernel,
        mesh=mesh,
        out_type=jax.ShapeDtypeStruct((B, D), jnp.float32),
        compiler_params=pltpu.CompilerParams(use_tc_tiling_on_sc=False),
        scratch_types=[
            pltpu.VMEM((n_steps, IDX_PER_STREAM), jnp.int32),
            pltpu.VMEM((2, rows_per_chunk, D), jnp.float32),
            pltpu.VMEM((2, CH, D), jnp.float32),
            pltpu.SemaphoreType.DMA,
            pltpu.SemaphoreType.DMA,
            pltpu.SemaphoreType.DMA,
        ],
    )
    def sc_kernel(ctx_hbm, table_hbm, out_hbm, idx_v, rows_v, ob_v,
                  sem0, sem1, out_sem):
        wid = lax.axis_index("s") * NC + lax.axis_index("c")
        pltpu.sync_copy(ctx_hbm.at[wid], idx_v)
        sems = (sem0, sem1)

        def gather_copies(c, par):
            for s in range(steps_per_chunk):
                step = c * steps_per_chunk + s
                yield pltpu.make_async_copy(
                    table_hbm.at[idx_v.at[step]],
                    rows_v.at[par].at[pl.ds(s * IDX_PER_STREAM, IDX_PER_STREAM)],
                    sems[par],
                )

        def fire(c, par):
            for cp in gather_copies(c, par):
                cp.start()

        def drain(c, par):
            for cp in gather_copies(c, par):
                cp.wait()

        def accumulate(c, par):
            @pl.when(c >= 2)
            def _():
                pltpu.make_async_copy(
                    ob_v.at[par], out_hbm.at[pl.ds(0, CH)], out_sem
                ).wait()

            def e_body(e, carry2):
                base = e * L
                for cg in range(cgroups):
                    sl = pl.ds(cg * LANES, LANES)
                    vals = [rows_v[par, base + j, sl] for j in range(L)]
                    ob_v[par, e, sl] = _tree_sum(vals) * inv_l
                return carry2

            lax.fori_loop(0, CH, e_body, 0, unroll=False)
            out_base = wid * b_per_w + c * CH
            pltpu.make_async_copy(
                ob_v.at[par], out_hbm.at[pl.ds(out_base, CH)], out_sem
            ).start()

        fire(0, 0)

        def pair_body(p, carry):
            c = p * 2
            fire(c + 1, 1)
            drain(c, 0)
            accumulate(c, 0)

            @pl.when(c + 2 < n_chunks)
            def _():
                fire(c + 2, 0)

            drain(c + 1, 1)
            accumulate(c + 1, 1)
            return carry

        lax.fori_loop(0, n_chunks // 2, pair_body, 0, unroll=False)
        for par in range(2):
            pltpu.make_async_copy(
                ob_v.at[par], out_hbm.at[pl.ds(0, CH)], out_sem
            ).wait()

    return sc_kernel


@jax.jit
def kernel(contexts, table):
    B, L = contexts.shape
    V, D = table.shape
    CH = 32
    assert (CH * L) % IDX_PER_STREAM == 0
    assert B % (NW * CH) == 0
    ctx3 = contexts.reshape(NW, (B // NW * L) // IDX_PER_STREAM, IDX_PER_STREAM)
    return _make_sc_kernel(B, L, V, D, CH)(ctx3, table)
